# baseline (speedup 1.0000x reference)
import functools

import jax
import jax.numpy as jnp
from jax.experimental import pallas as pl
from jax.experimental.pallas import tpu as pltpu

_NB = 128


def _bc(v):
    return v[:, :, None, None]


def _first_conv_pool_kernel(c1, x_ref, w_ref, s_ref, b_ref, out_ref):
    nb = out_ref.shape[-1]
    v = x_ref[...]
    r2 = v.reshape(64, 114, 2 * nb)
    e2 = r2[:, :, :nb].reshape(64, 57, 2 * nb)
    o2 = r2[:, :, nb:].reshape(64, 57, 2 * nb)
    cu = [e2[:, :, :nb].reshape(16, 4, 57, nb),
          o2[:, :, :nb].reshape(16, 4, 57, nb),
          e2[:, :, nb:].reshape(16, 4, 57, nb),
          o2[:, :, nb:].reshape(16, 4, 57, nb)]
    for co in range(c1):
        sv = s_ref[co, pl.ds(0, 1)]
        bv = b_ref[co, pl.ds(0, 1)]
        m = None
        for dy in (0, 1):
            for dx in (0, 1):
                acc = jnp.zeros((14, 56, nb), jnp.float32)
                for kh in range(5):
                    t = 2 * dy + kh
                    for kw in range(5):
                        u = 2 * dx + kw
                        slab = cu[u % 4][t // 4:t // 4 + 14, t % 4,
                                         u // 4:u // 4 + 56, :]
                        acc = acc + slab * w_ref[co, pl.ds(kh * 5 + kw, 1)]
                vq = jnp.clip(acc * sv + bv, 0.0, 6.0)
                m = vq if m is None else jnp.maximum(m, vq)
        out_ref[co] = m


def _first_stage(x_nchw, w_row, s_row, b_row):
    n = x_nchw.shape[0]
    nb = _NB if n % _NB == 0 else n
    c1 = s_row.shape[1]
    xt = jnp.transpose(x_nchw.reshape(n, 224 * 224), (1, 0))
    xpp = jnp.pad(xt.reshape(224, 224, n), ((2, 6), (2, 2), (0, 0)))
    bands = jnp.concatenate([xpp[56 * r:56 * r + 64] for r in range(4)],
                            axis=0)
    return pl.pallas_call(
        functools.partial(_first_conv_pool_kernel, c1),
        out_shape=jax.ShapeDtypeStruct((c1, 56, 56, n), jnp.float32),
        grid=(n // nb if n % _NB == 0 else 1, 4),
        in_specs=[
            pl.BlockSpec((64, 228, nb), lambda i, r: (r, 0, i)),
            pl.BlockSpec((c1, 25), lambda i, r: (0, 0)),
            pl.BlockSpec((c1, 1), lambda i, r: (0, 0)),
            pl.BlockSpec((c1, 1), lambda i, r: (0, 0)),
        ],
        out_specs=pl.BlockSpec((c1, 14, 56, nb), lambda i, r: (0, r, 0, i)),
        compiler_params=pltpu.CompilerParams(
            dimension_semantics=("parallel", "parallel")),
    )(bands, w_row.reshape(c1, 25), s_row.reshape(c1, 1),
      b_row.reshape(c1, 1))


def _resblock_body(cin, cout, ho, stride, with_fc, *refs):
    if with_fc:
        (x_ref, w1_ref, s1_ref, b1_ref, w2_ref, s2_ref, b2_ref,
         wb_ref, sb_ref, bb_ref, fcw_ref, out_ref, x2_ref) = refs
    else:
        (x_ref, w1_ref, s1_ref, b1_ref, w2_ref, s2_ref, b2_ref,
         wb_ref, sb_ref, bb_ref, out_ref, x2_ref) = refs
    nb = out_ref.shape[-1]

    def tap_s2(kh, kw, ci):
        ph = 2 * (kh % 2) + (kw % 2)
        return x_ref[ph * cin + ci, pl.ds(kh // 2, ho), pl.ds(kw // 2, ho), :]

    def tap_s1_in(kh, kw, ci):
        return x_ref[ci, pl.ds(kh, ho), pl.ds(kw, ho), :]

    def tap_scratch(kh, kw, ci):
        return x2_ref[ci, pl.ds(kh, ho), pl.ds(kw, ho), :]

    def conv3_to(tap, n_ci, w_ref, s_ref, b_ref, co):
        acc = jnp.zeros((ho, ho, nb), jnp.float32)
        for kh in range(3):
            for kw in range(3):
                base = (kh * 3 + kw) * n_ci
                for ci in range(n_ci):
                    acc = acc + tap(kh, kw, ci) * w_ref[co, pl.ds(base + ci, 1)]
        return jnp.clip(acc * s_ref[co, pl.ds(0, 1)]
                        + b_ref[co, pl.ds(0, 1)], 0.0, 6.0)

    x2_ref[...] = jnp.zeros(x2_ref.shape, jnp.float32)
    tap1 = tap_s2 if stride == 2 else tap_s1_in
    for co in range(cout):
        x2_ref[co, pl.ds(1, ho), pl.ds(1, ho), :] = conv3_to(
            tap1, cin, w1_ref, s1_ref, b1_ref, co)

    if with_fc:
        s0 = jnp.zeros((nb,), jnp.float32)
        s1v = jnp.zeros((nb,), jnp.float32)
    for co in range(cout):
        y2 = conv3_to(tap_scratch, cout, w2_ref, s2_ref, b2_ref, co)
        accb = jnp.zeros((ho, ho, nb), jnp.float32)
        for ci in range(cin):
            if stride == 2:
                slab = x_ref[3 * cin + ci, pl.ds(0, ho), pl.ds(0, ho), :]
            else:
                slab = x_ref[ci, pl.ds(1, ho), pl.ds(1, ho), :]
            accb = accb + slab * wb_ref[co, pl.ds(ci, 1)]
        yb = jnp.clip(accb * sb_ref[co, pl.ds(0, 1)]
                      + bb_ref[co, pl.ds(0, 1)], 0.0, 6.0)
        y = y2 + yb
        if with_fc:
            yr = y.astype(jnp.bfloat16).astype(jnp.float32)
            w0 = fcw_ref[co, :, :, pl.ds(0, 1)].astype(jnp.bfloat16).astype(jnp.float32)
            w1 = fcw_ref[co, :, :, pl.ds(1, 1)].astype(jnp.bfloat16).astype(jnp.float32)
            s0 = s0 + jnp.sum(yr * w0, axis=(0, 1))
            s1v = s1v + jnp.sum(yr * w1, axis=(0, 1))
        else:
            out_ref[co] = y
    if with_fc:
        out_ref[pl.ds(0, 1), :] = s0[None]
        out_ref[pl.ds(1, 1), :] = jax.nn.sigmoid(s1v)[None]


def _resblock(x, bp, stride, fcw=None):
    cin, h, _, n = x.shape
    nb = _NB if n % _NB == 0 else n
    cout = bp['s1'].shape[0]
    ho = h // stride
    xp = jnp.pad(x, ((0, 0), (1, 1), (1, 1), (0, 0)))
    if stride == 2:
        hp = (h + 2) // 2
        x_in = jnp.transpose(xp.reshape(cin, hp, 2, hp, 2, n),
                             (2, 4, 0, 1, 3, 5)).reshape(4 * cin, hp, hp, n)
        in_spec = pl.BlockSpec((4 * cin, hp, hp, nb), lambda i: (0, 0, 0, i))
    else:
        x_in = xp
        in_spec = pl.BlockSpec((cin, h + 2, h + 2, nb), lambda i: (0, 0, 0, i))

    def vspec(a):
        return pl.BlockSpec(a.shape, lambda i, _r=a.ndim: (0,) * _r)

    with_fc = fcw is not None
    wargs = [bp['w1'], bp['s1'], bp['b1'], bp['w2'], bp['s2'], bp['b2'],
             bp['wb'], bp['sb'], bp['bb']]
    if with_fc:
        wargs.append(fcw)
        out_shape = jax.ShapeDtypeStruct((2, n), jnp.float32)
        out_spec = pl.BlockSpec((2, nb), lambda i: (0, i))
    else:
        out_shape = jax.ShapeDtypeStruct((cout, ho, ho, n), jnp.float32)
        out_spec = pl.BlockSpec((cout, ho, ho, nb), lambda i: (0, 0, 0, i))
    return pl.pallas_call(
        functools.partial(_resblock_body, cin, cout, ho, stride, with_fc),
        out_shape=out_shape,
        grid=(n // nb if n % _NB == 0 else 1,),
        in_specs=[in_spec] + [vspec(a) for a in wargs],
        out_specs=out_spec,
        scratch_shapes=[pltpu.VMEM((cout, ho + 2, ho + 2, nb), jnp.float32)],
        compiler_params=pltpu.CompilerParams(
            dimension_semantics=("parallel",)),
    )(x_in, *wargs)


def kernel(x_nchw, first_w, first_s, first_b,
           rb1a_w1, rb1a_s1, rb1a_b1, rb1a_w2, rb1a_s2, rb1a_b2, rb1a_wb, rb1a_sb, rb1a_bb,
           rb1b_w1, rb1b_s1, rb1b_b1, rb1b_w2, rb1b_s2, rb1b_b2, rb1b_wb, rb1b_sb, rb1b_bb,
           rb2a_w1, rb2a_s1, rb2a_b1, rb2a_w2, rb2a_s2, rb2a_b2, rb2a_wb, rb2a_sb, rb2a_bb,
           rb2b_w1, rb2b_s1, rb2b_b1, rb2b_w2, rb2b_s2, rb2b_b2, rb2b_wb, rb2b_sb, rb2b_bb,
           rb3a_w1, rb3a_s1, rb3a_b1, rb3a_w2, rb3a_s2, rb3a_b2, rb3a_wb, rb3a_sb, rb3a_bb,
           rb3b_w1, rb3b_s1, rb3b_b1, rb3b_w2, rb3b_s2, rb3b_b2, rb3b_wb, rb3b_sb, rb3b_bb,
           fc_w):
    x = _first_stage(x_nchw, first_w, first_s, first_b)
    blocks = [
        (dict(w1=rb1a_w1, s1=rb1a_s1, b1=rb1a_b1, w2=rb1a_w2, s2=rb1a_s2,
              b2=rb1a_b2, wb=rb1a_wb, sb=rb1a_sb, bb=rb1a_bb), 2),
        (dict(w1=rb1b_w1, s1=rb1b_s1, b1=rb1b_b1, w2=rb1b_w2, s2=rb1b_s2,
              b2=rb1b_b2, wb=rb1b_wb, sb=rb1b_sb, bb=rb1b_bb), 1),
        (dict(w1=rb2a_w1, s1=rb2a_s1, b1=rb2a_b1, w2=rb2a_w2, s2=rb2a_s2,
              b2=rb2a_b2, wb=rb2a_wb, sb=rb2a_sb, bb=rb2a_bb), 2),
        (dict(w1=rb2b_w1, s1=rb2b_s1, b1=rb2b_b1, w2=rb2b_w2, s2=rb2b_s2,
              b2=rb2b_b2, wb=rb2b_wb, sb=rb2b_sb, bb=rb2b_bb), 1),
        (dict(w1=rb3a_w1, s1=rb3a_s1, b1=rb3a_b1, w2=rb3a_w2, s2=rb3a_s2,
              b2=rb3a_b2, wb=rb3a_wb, sb=rb3a_sb, bb=rb3a_bb), 2),
    ]
    for bp, stride in blocks:
        x = _resblock(x, bp, stride)
    c4 = rb3b_s1.shape[0]
    fcw = fc_w.reshape(7, c4, 7, 2).transpose(1, 0, 2, 3)
    out = _resblock(x, dict(w1=rb3b_w1, s1=rb3b_s1, b1=rb3b_b1, w2=rb3b_w2,
                            s2=rb3b_s2, b2=rb3b_b2, wb=rb3b_wb, sb=rb3b_sb,
                            bb=rb3b_bb), 1, fcw=fcw)
    return [out[0], out[1]]


# speedup vs baseline: 1.6888x; 1.6888x over previous
import functools

import jax
import jax.numpy as jnp
from jax.experimental import pallas as pl
from jax.experimental.pallas import tpu as pltpu

_NB = 128


def _bc(v):
    return v[:, :, None, None]


def _first_conv_pool_kernel(c1, ph_ref, w_ref, s_ref, b_ref, out_ref):
    nb = out_ref.shape[-1]
    for co in range(c1):
        sv = s_ref[co, pl.ds(0, 1)]
        bv = b_ref[co, pl.ds(0, 1)]
        m = None
        for dy in (0, 1):
            for dx in (0, 1):
                acc = jnp.zeros((14, 56, nb), jnp.float32)
                for kh in range(5):
                    t = 2 * dy + kh
                    for kw in range(5):
                        u = 2 * dx + kw
                        slab = ph_ref[4 * (t % 4) + (u % 4),
                                      pl.ds(t // 4, 14), pl.ds(u // 4, 56), :]
                        acc = acc + slab * w_ref[co, pl.ds(kh * 5 + kw, 1)]
                v = jnp.clip(acc * sv + bv, 0.0, 6.0)
                m = v if m is None else jnp.maximum(m, v)
        out_ref[co] = m


def _first_stage(x_nchw, w_row, s_row, b_row):
    n = x_nchw.shape[0]
    nb = _NB if n % _NB == 0 else n
    c1 = s_row.shape[1]
    xp = jnp.pad(x_nchw[:, 0], ((0, 0), (2, 2), (2, 2)))
    ph16 = jnp.transpose(xp.reshape(n, 57, 4, 57, 4),
                         (2, 4, 1, 3, 0)).reshape(16, 57, 57, n)
    bands = jnp.concatenate([ph16[:, 14 * r:14 * r + 15] for r in range(4)],
                            axis=0)
    return pl.pallas_call(
        functools.partial(_first_conv_pool_kernel, c1),
        out_shape=jax.ShapeDtypeStruct((c1, 56, 56, n), jnp.float32),
        grid=(n // nb if n % _NB == 0 else 1, 4),
        in_specs=[
            pl.BlockSpec((16, 15, 57, nb), lambda i, r: (r, 0, 0, i)),
            pl.BlockSpec((c1, 25), lambda i, r: (0, 0)),
            pl.BlockSpec((c1, 1), lambda i, r: (0, 0)),
            pl.BlockSpec((c1, 1), lambda i, r: (0, 0)),
        ],
        out_specs=pl.BlockSpec((c1, 14, 56, nb), lambda i, r: (0, r, 0, i)),
        compiler_params=pltpu.CompilerParams(
            dimension_semantics=("parallel", "parallel")),
    )(bands, w_row.reshape(c1, 25), s_row.reshape(c1, 1),
      b_row.reshape(c1, 1))


def _resblock_body(cin, cout, ho, stride, with_fc, *refs):
    if with_fc:
        (x_ref, w1_ref, s1_ref, b1_ref, w2_ref, s2_ref, b2_ref,
         wb_ref, sb_ref, bb_ref, fcw_ref, out_ref, x2_ref) = refs
    else:
        (x_ref, w1_ref, s1_ref, b1_ref, w2_ref, s2_ref, b2_ref,
         wb_ref, sb_ref, bb_ref, out_ref, x2_ref) = refs
    nb = out_ref.shape[-1]

    def tap_s2(kh, kw, ci):
        ph = 2 * (kh % 2) + (kw % 2)
        return x_ref[ph * cin + ci, pl.ds(kh // 2, ho), pl.ds(kw // 2, ho), :]

    def tap_s1_in(kh, kw, ci):
        return x_ref[ci, pl.ds(kh, ho), pl.ds(kw, ho), :]

    def tap_scratch(kh, kw, ci):
        return x2_ref[ci, pl.ds(kh, ho), pl.ds(kw, ho), :]

    def conv3_to(tap, n_ci, w_ref, s_ref, b_ref, co):
        acc = jnp.zeros((ho, ho, nb), jnp.float32)
        for kh in range(3):
            for kw in range(3):
                base = (kh * 3 + kw) * n_ci
                for ci in range(n_ci):
                    acc = acc + tap(kh, kw, ci) * w_ref[co, pl.ds(base + ci, 1)]
        return jnp.clip(acc * s_ref[co, pl.ds(0, 1)]
                        + b_ref[co, pl.ds(0, 1)], 0.0, 6.0)

    x2_ref[...] = jnp.zeros(x2_ref.shape, jnp.float32)
    tap1 = tap_s2 if stride == 2 else tap_s1_in
    for co in range(cout):
        x2_ref[co, pl.ds(1, ho), pl.ds(1, ho), :] = conv3_to(
            tap1, cin, w1_ref, s1_ref, b1_ref, co)

    if with_fc:
        s0 = jnp.zeros((nb,), jnp.float32)
        s1v = jnp.zeros((nb,), jnp.float32)
    for co in range(cout):
        y2 = conv3_to(tap_scratch, cout, w2_ref, s2_ref, b2_ref, co)
        accb = jnp.zeros((ho, ho, nb), jnp.float32)
        for ci in range(cin):
            if stride == 2:
                slab = x_ref[3 * cin + ci, pl.ds(0, ho), pl.ds(0, ho), :]
            else:
                slab = x_ref[ci, pl.ds(1, ho), pl.ds(1, ho), :]
            accb = accb + slab * wb_ref[co, pl.ds(ci, 1)]
        yb = jnp.clip(accb * sb_ref[co, pl.ds(0, 1)]
                      + bb_ref[co, pl.ds(0, 1)], 0.0, 6.0)
        y = y2 + yb
        if with_fc:
            yr = y.astype(jnp.bfloat16).astype(jnp.float32)
            w0 = fcw_ref[co, :, :, pl.ds(0, 1)].astype(jnp.bfloat16).astype(jnp.float32)
            w1 = fcw_ref[co, :, :, pl.ds(1, 1)].astype(jnp.bfloat16).astype(jnp.float32)
            s0 = s0 + jnp.sum(yr * w0, axis=(0, 1))
            s1v = s1v + jnp.sum(yr * w1, axis=(0, 1))
        else:
            out_ref[co] = y
    if with_fc:
        out_ref[pl.ds(0, 1), :] = s0[None]
        out_ref[pl.ds(1, 1), :] = jax.nn.sigmoid(s1v)[None]


def _resblock(x, bp, stride, fcw=None):
    cin, h, _, n = x.shape
    nb = _NB if n % _NB == 0 else n
    cout = bp['s1'].shape[0]
    ho = h // stride
    xp = jnp.pad(x, ((0, 0), (1, 1), (1, 1), (0, 0)))
    if stride == 2:
        hp = (h + 2) // 2
        x_in = jnp.transpose(xp.reshape(cin, hp, 2, hp, 2, n),
                             (2, 4, 0, 1, 3, 5)).reshape(4 * cin, hp, hp, n)
        in_spec = pl.BlockSpec((4 * cin, hp, hp, nb), lambda i: (0, 0, 0, i))
    else:
        x_in = xp
        in_spec = pl.BlockSpec((cin, h + 2, h + 2, nb), lambda i: (0, 0, 0, i))

    def vspec(a):
        return pl.BlockSpec(a.shape, lambda i, _r=a.ndim: (0,) * _r)

    with_fc = fcw is not None
    wargs = [bp['w1'], bp['s1'], bp['b1'], bp['w2'], bp['s2'], bp['b2'],
             bp['wb'], bp['sb'], bp['bb']]
    if with_fc:
        wargs.append(fcw)
        out_shape = jax.ShapeDtypeStruct((2, n), jnp.float32)
        out_spec = pl.BlockSpec((2, nb), lambda i: (0, i))
    else:
        out_shape = jax.ShapeDtypeStruct((cout, ho, ho, n), jnp.float32)
        out_spec = pl.BlockSpec((cout, ho, ho, nb), lambda i: (0, 0, 0, i))
    return pl.pallas_call(
        functools.partial(_resblock_body, cin, cout, ho, stride, with_fc),
        out_shape=out_shape,
        grid=(n // nb if n % _NB == 0 else 1,),
        in_specs=[in_spec] + [vspec(a) for a in wargs],
        out_specs=out_spec,
        scratch_shapes=[pltpu.VMEM((cout, ho + 2, ho + 2, nb), jnp.float32)],
        compiler_params=pltpu.CompilerParams(
            dimension_semantics=("parallel",)),
    )(x_in, *wargs)


def kernel(x_nchw, first_w, first_s, first_b,
           rb1a_w1, rb1a_s1, rb1a_b1, rb1a_w2, rb1a_s2, rb1a_b2, rb1a_wb, rb1a_sb, rb1a_bb,
           rb1b_w1, rb1b_s1, rb1b_b1, rb1b_w2, rb1b_s2, rb1b_b2, rb1b_wb, rb1b_sb, rb1b_bb,
           rb2a_w1, rb2a_s1, rb2a_b1, rb2a_w2, rb2a_s2, rb2a_b2, rb2a_wb, rb2a_sb, rb2a_bb,
           rb2b_w1, rb2b_s1, rb2b_b1, rb2b_w2, rb2b_s2, rb2b_b2, rb2b_wb, rb2b_sb, rb2b_bb,
           rb3a_w1, rb3a_s1, rb3a_b1, rb3a_w2, rb3a_s2, rb3a_b2, rb3a_wb, rb3a_sb, rb3a_bb,
           rb3b_w1, rb3b_s1, rb3b_b1, rb3b_w2, rb3b_s2, rb3b_b2, rb3b_wb, rb3b_sb, rb3b_bb,
           fc_w):
    x = _first_stage(x_nchw, first_w, first_s, first_b)
    blocks = [
        (dict(w1=rb1a_w1, s1=rb1a_s1, b1=rb1a_b1, w2=rb1a_w2, s2=rb1a_s2,
              b2=rb1a_b2, wb=rb1a_wb, sb=rb1a_sb, bb=rb1a_bb), 2),
        (dict(w1=rb1b_w1, s1=rb1b_s1, b1=rb1b_b1, w2=rb1b_w2, s2=rb1b_s2,
              b2=rb1b_b2, wb=rb1b_wb, sb=rb1b_sb, bb=rb1b_bb), 1),
        (dict(w1=rb2a_w1, s1=rb2a_s1, b1=rb2a_b1, w2=rb2a_w2, s2=rb2a_s2,
              b2=rb2a_b2, wb=rb2a_wb, sb=rb2a_sb, bb=rb2a_bb), 2),
        (dict(w1=rb2b_w1, s1=rb2b_s1, b1=rb2b_b1, w2=rb2b_w2, s2=rb2b_s2,
              b2=rb2b_b2, wb=rb2b_wb, sb=rb2b_sb, bb=rb2b_bb), 1),
        (dict(w1=rb3a_w1, s1=rb3a_s1, b1=rb3a_b1, w2=rb3a_w2, s2=rb3a_s2,
              b2=rb3a_b2, wb=rb3a_wb, sb=rb3a_sb, bb=rb3a_bb), 2),
    ]
    for bp, stride in blocks:
        x = _resblock(x, bp, stride)
    c4 = rb3b_s1.shape[0]
    fcw = fc_w.reshape(7, c4, 7, 2).transpose(1, 0, 2, 3)
    out = _resblock(x, dict(w1=rb3b_w1, s1=rb3b_s1, b1=rb3b_b1, w2=rb3b_w2,
                            s2=rb3b_s2, b2=rb3b_b2, wb=rb3b_wb, sb=rb3b_sb,
                            bb=rb3b_bb), 1, fcw=fcw)
    return [out[0], out[1]]


# speedup vs baseline: 1.9172x; 1.1352x over previous
import functools

import jax
import jax.numpy as jnp
from jax.experimental import pallas as pl
from jax.experimental.pallas import tpu as pltpu

_NB = 128


def _bc(v):
    return v[:, :, None, None]


def _first_conv_pool_kernel(c1, pha_ref, phb_ref, w_ref, s_ref, b_ref,
                            out_ref, ph_ref):
    nb = out_ref.shape[-1]
    ph_ref[:, pl.ds(0, 14)] = pha_ref[...]
    ph_ref[:, pl.ds(14, 1)] = phb_ref[...]
    for co in range(c1):
        sv = s_ref[co, pl.ds(0, 1)]
        bv = b_ref[co, pl.ds(0, 1)]
        m = None
        for dy in (0, 1):
            for dx in (0, 1):
                acc = jnp.zeros((14, 56, nb), jnp.float32)
                for kh in range(5):
                    t = 2 * dy + kh
                    for kw in range(5):
                        u = 2 * dx + kw
                        slab = ph_ref[4 * (t % 4) + (u % 4),
                                      pl.ds(t // 4, 14), pl.ds(u // 4, 56), :]
                        acc = acc + slab * w_ref[co, pl.ds(kh * 5 + kw, 1)]
                v = jnp.clip(acc * sv + bv, 0.0, 6.0)
                m = v if m is None else jnp.maximum(m, v)
        out_ref[co] = m


def _first_stage(x_nchw, w_row, s_row, b_row):
    n = x_nchw.shape[0]
    nb = _NB if n % _NB == 0 else n
    c1 = s_row.shape[1]
    xp = jnp.pad(x_nchw[:, 0], ((0, 0), (2, 2), (2, 2)))
    ph16 = jnp.transpose(xp.reshape(n, 57, 4, 57, 4),
                         (2, 4, 1, 3, 0)).reshape(16, 57, 57, n)
    return pl.pallas_call(
        functools.partial(_first_conv_pool_kernel, c1),
        out_shape=jax.ShapeDtypeStruct((c1, 56, 56, n), jnp.float32),
        grid=(n // nb if n % _NB == 0 else 1, 4),
        in_specs=[
            pl.BlockSpec((16, 14, 57, nb), lambda i, r: (0, r, 0, i)),
            pl.BlockSpec((16, 1, 57, nb), lambda i, r: (0, 14 * r + 14, 0, i)),
            pl.BlockSpec((c1, 25), lambda i, r: (0, 0)),
            pl.BlockSpec((c1, 1), lambda i, r: (0, 0)),
            pl.BlockSpec((c1, 1), lambda i, r: (0, 0)),
        ],
        out_specs=pl.BlockSpec((c1, 14, 56, nb), lambda i, r: (0, r, 0, i)),
        scratch_shapes=[pltpu.VMEM((16, 15, 57, nb), jnp.float32)],
        compiler_params=pltpu.CompilerParams(
            dimension_semantics=("parallel", "parallel")),
    )(ph16, ph16, w_row.reshape(c1, 25), s_row.reshape(c1, 1),
      b_row.reshape(c1, 1))


def _resblock_body(cin, cout, ho, stride, with_fc, *refs):
    if with_fc:
        (x_ref, w1_ref, s1_ref, b1_ref, w2_ref, s2_ref, b2_ref,
         wb_ref, sb_ref, bb_ref, fcw_ref, out_ref, x2_ref) = refs
    else:
        (x_ref, w1_ref, s1_ref, b1_ref, w2_ref, s2_ref, b2_ref,
         wb_ref, sb_ref, bb_ref, out_ref, x2_ref) = refs
    nb = out_ref.shape[-1]

    def tap_s2(kh, kw, ci):
        ph = 2 * (kh % 2) + (kw % 2)
        return x_ref[ph * cin + ci, pl.ds(kh // 2, ho), pl.ds(kw // 2, ho), :]

    def tap_s1_in(kh, kw, ci):
        return x_ref[ci, pl.ds(kh, ho), pl.ds(kw, ho), :]

    def tap_scratch(kh, kw, ci):
        return x2_ref[ci, pl.ds(kh, ho), pl.ds(kw, ho), :]

    def conv3_to(tap, n_ci, w_ref, s_ref, b_ref, co):
        acc = jnp.zeros((ho, ho, nb), jnp.float32)
        for kh in range(3):
            for kw in range(3):
                base = (kh * 3 + kw) * n_ci
                for ci in range(n_ci):
                    acc = acc + tap(kh, kw, ci) * w_ref[co, pl.ds(base + ci, 1)]
        return jnp.clip(acc * s_ref[co, pl.ds(0, 1)]
                        + b_ref[co, pl.ds(0, 1)], 0.0, 6.0)

    x2_ref[...] = jnp.zeros(x2_ref.shape, jnp.float32)
    tap1 = tap_s2 if stride == 2 else tap_s1_in
    for co in range(cout):
        x2_ref[co, pl.ds(1, ho), pl.ds(1, ho), :] = conv3_to(
            tap1, cin, w1_ref, s1_ref, b1_ref, co)

    if with_fc:
        s0 = jnp.zeros((nb,), jnp.float32)
        s1v = jnp.zeros((nb,), jnp.float32)
    for co in range(cout):
        y2 = conv3_to(tap_scratch, cout, w2_ref, s2_ref, b2_ref, co)
        accb = jnp.zeros((ho, ho, nb), jnp.float32)
        for ci in range(cin):
            if stride == 2:
                slab = x_ref[3 * cin + ci, pl.ds(0, ho), pl.ds(0, ho), :]
            else:
                slab = x_ref[ci, pl.ds(1, ho), pl.ds(1, ho), :]
            accb = accb + slab * wb_ref[co, pl.ds(ci, 1)]
        yb = jnp.clip(accb * sb_ref[co, pl.ds(0, 1)]
                      + bb_ref[co, pl.ds(0, 1)], 0.0, 6.0)
        y = y2 + yb
        if with_fc:
            yr = y.astype(jnp.bfloat16).astype(jnp.float32)
            w0 = fcw_ref[co, :, :, pl.ds(0, 1)].astype(jnp.bfloat16).astype(jnp.float32)
            w1 = fcw_ref[co, :, :, pl.ds(1, 1)].astype(jnp.bfloat16).astype(jnp.float32)
            s0 = s0 + jnp.sum(yr * w0, axis=(0, 1))
            s1v = s1v + jnp.sum(yr * w1, axis=(0, 1))
        else:
            out_ref[co] = y
    if with_fc:
        out_ref[pl.ds(0, 1), :] = s0[None]
        out_ref[pl.ds(1, 1), :] = jax.nn.sigmoid(s1v)[None]


def _resblock(x, bp, stride, fcw=None):
    cin, h, _, n = x.shape
    nb = _NB if n % _NB == 0 else n
    cout = bp['s1'].shape[0]
    ho = h // stride
    xp = jnp.pad(x, ((0, 0), (1, 1), (1, 1), (0, 0)))
    if stride == 2:
        hp = (h + 2) // 2
        x_in = jnp.transpose(xp.reshape(cin, hp, 2, hp, 2, n),
                             (2, 4, 0, 1, 3, 5)).reshape(4 * cin, hp, hp, n)
        in_spec = pl.BlockSpec((4 * cin, hp, hp, nb), lambda i: (0, 0, 0, i))
    else:
        x_in = xp
        in_spec = pl.BlockSpec((cin, h + 2, h + 2, nb), lambda i: (0, 0, 0, i))

    def vspec(a):
        return pl.BlockSpec(a.shape, lambda i, _r=a.ndim: (0,) * _r)

    with_fc = fcw is not None
    wargs = [bp['w1'], bp['s1'], bp['b1'], bp['w2'], bp['s2'], bp['b2'],
             bp['wb'], bp['sb'], bp['bb']]
    if with_fc:
        wargs.append(fcw)
        out_shape = jax.ShapeDtypeStruct((2, n), jnp.float32)
        out_spec = pl.BlockSpec((2, nb), lambda i: (0, i))
    else:
        out_shape = jax.ShapeDtypeStruct((cout, ho, ho, n), jnp.float32)
        out_spec = pl.BlockSpec((cout, ho, ho, nb), lambda i: (0, 0, 0, i))
    return pl.pallas_call(
        functools.partial(_resblock_body, cin, cout, ho, stride, with_fc),
        out_shape=out_shape,
        grid=(n // nb if n % _NB == 0 else 1,),
        in_specs=[in_spec] + [vspec(a) for a in wargs],
        out_specs=out_spec,
        scratch_shapes=[pltpu.VMEM((cout, ho + 2, ho + 2, nb), jnp.float32)],
        compiler_params=pltpu.CompilerParams(
            dimension_semantics=("parallel",)),
    )(x_in, *wargs)


def kernel(x_nchw, first_w, first_s, first_b,
           rb1a_w1, rb1a_s1, rb1a_b1, rb1a_w2, rb1a_s2, rb1a_b2, rb1a_wb, rb1a_sb, rb1a_bb,
           rb1b_w1, rb1b_s1, rb1b_b1, rb1b_w2, rb1b_s2, rb1b_b2, rb1b_wb, rb1b_sb, rb1b_bb,
           rb2a_w1, rb2a_s1, rb2a_b1, rb2a_w2, rb2a_s2, rb2a_b2, rb2a_wb, rb2a_sb, rb2a_bb,
           rb2b_w1, rb2b_s1, rb2b_b1, rb2b_w2, rb2b_s2, rb2b_b2, rb2b_wb, rb2b_sb, rb2b_bb,
           rb3a_w1, rb3a_s1, rb3a_b1, rb3a_w2, rb3a_s2, rb3a_b2, rb3a_wb, rb3a_sb, rb3a_bb,
           rb3b_w1, rb3b_s1, rb3b_b1, rb3b_w2, rb3b_s2, rb3b_b2, rb3b_wb, rb3b_sb, rb3b_bb,
           fc_w):
    x = _first_stage(x_nchw, first_w, first_s, first_b)
    blocks = [
        (dict(w1=rb1a_w1, s1=rb1a_s1, b1=rb1a_b1, w2=rb1a_w2, s2=rb1a_s2,
              b2=rb1a_b2, wb=rb1a_wb, sb=rb1a_sb, bb=rb1a_bb), 2),
        (dict(w1=rb1b_w1, s1=rb1b_s1, b1=rb1b_b1, w2=rb1b_w2, s2=rb1b_s2,
              b2=rb1b_b2, wb=rb1b_wb, sb=rb1b_sb, bb=rb1b_bb), 1),
        (dict(w1=rb2a_w1, s1=rb2a_s1, b1=rb2a_b1, w2=rb2a_w2, s2=rb2a_s2,
              b2=rb2a_b2, wb=rb2a_wb, sb=rb2a_sb, bb=rb2a_bb), 2),
        (dict(w1=rb2b_w1, s1=rb2b_s1, b1=rb2b_b1, w2=rb2b_w2, s2=rb2b_s2,
              b2=rb2b_b2, wb=rb2b_wb, sb=rb2b_sb, bb=rb2b_bb), 1),
        (dict(w1=rb3a_w1, s1=rb3a_s1, b1=rb3a_b1, w2=rb3a_w2, s2=rb3a_s2,
              b2=rb3a_b2, wb=rb3a_wb, sb=rb3a_sb, bb=rb3a_bb), 2),
    ]
    for bp, stride in blocks:
        x = _resblock(x, bp, stride)
    c4 = rb3b_s1.shape[0]
    fcw = fc_w.reshape(7, c4, 7, 2).transpose(1, 0, 2, 3)
    out = _resblock(x, dict(w1=rb3b_w1, s1=rb3b_s1, b1=rb3b_b1, w2=rb3b_w2,
                            s2=rb3b_s2, b2=rb3b_b2, wb=rb3b_wb, sb=rb3b_sb,
                            bb=rb3b_bb), 1, fcw=fcw)
    return [out[0], out[1]]
